# Initial kernel scaffold; baseline (speedup 1.0000x reference)
import jax
import jax.numpy as jnp
from jax.experimental import pallas as pl


def kernel(x, edge_index, batch, W1, b1, W2, b2, Wc, bc):
    raise NotImplementedError("write your pallas kernel here")


# speedup vs baseline: 13.8936x; 13.8936x over previous
import functools

import jax
import jax.numpy as jnp
from jax import lax
from jax.experimental import pallas as pl
from jax.experimental.pallas import tpu as pltpu
from jax.experimental.pallas import tpu_sc as plsc

N = 10000
E = 320000
D_IN = 128
H = 64
OUT = 2
G = 128

NTILES = 32
CHUNK = 128
NCHUNK = 80
E_PAD = NTILES * NCHUNK * CHUNK
N_PAD = 10112
RPT = N_PAD // 16
DEG_W = 16

BLK = 1000
NB = N // BLK


@functools.cache
def _sc_degree_call():
    mesh = plsc.VectorSubcoreMesh(core_axis_name="c", subcore_axis_name="s")
    return pl.kernel(
        _sc_degree,
        out_type=[jax.ShapeDtypeStruct((N_PAD, DEG_W), jnp.float32),
                  jax.ShapeDtypeStruct((N_PAD, DEG_W), jnp.float32)],
        mesh=mesh,
        scratch_types=[
            pltpu.VMEM((NCHUNK, CHUNK), jnp.int32),
            pltpu.VMEM((CHUNK, DEG_W), jnp.float32),
            pltpu.VMEM_SHARED((N_PAD, DEG_W), jnp.float32),
        ],
        compiler_params=pltpu.CompilerParams(use_tc_tiling_on_sc=False),
    )


def _sc_degree(dst_hbm, z16_hbm, deg0_hbm, deg1_hbm, dst_v, ones_v, acc_sh):
    cid = lax.axis_index("c")
    sid = lax.axis_index("s")
    wid = sid * 2 + cid
    r0 = sid * RPT
    for r in range(CHUNK):
        ones_v[r] = jnp.ones((16,), jnp.float32)
    pltpu.sync_copy(z16_hbm.at[pl.ds(r0, RPT)], acc_sh.at[pl.ds(r0, RPT)])
    pltpu.sync_copy(dst_hbm.at[wid], dst_v)
    plsc.subcore_barrier()

    @pl.loop(0, NCHUNK)
    def _(j):
        pltpu.sync_copy(ones_v, acc_sh.at[dst_v.at[j]], add=True)

    plsc.subcore_barrier()

    @pl.when(cid == 0)
    def _():
        pltpu.sync_copy(acc_sh.at[pl.ds(r0, RPT)], deg0_hbm.at[pl.ds(r0, RPT)])

    @pl.when(cid == 1)
    def _():
        pltpu.sync_copy(acc_sh.at[pl.ds(r0, RPT)], deg1_hbm.at[pl.ds(r0, RPT)])


@functools.cache
def _sc_edge_call():
    mesh = plsc.VectorSubcoreMesh(core_axis_name="c", subcore_axis_name="s")
    return pl.kernel(
        _sc_edge,
        out_type=[jax.ShapeDtypeStruct((N_PAD, H), jnp.float32),
                  jax.ShapeDtypeStruct((N_PAD, H), jnp.float32)],
        mesh=mesh,
        scratch_types=[
            pltpu.VMEM((NCHUNK, CHUNK), jnp.int32),
            pltpu.VMEM((NCHUNK, CHUNK), jnp.int32),
            pltpu.VMEM((CHUNK, H), jnp.float32),
            pltpu.VMEM_SHARED((N_PAD, H), jnp.float32),
            pltpu.SemaphoreType.DMA,
        ],
        compiler_params=pltpu.CompilerParams(use_tc_tiling_on_sc=False),
    )


def _sc_edge(src_hbm, dst_hbm, g_hbm, z64_hbm, acc0_hbm, acc1_hbm,
             src_v, dst_v, rows_v, acc_sh, sem):
    cid = lax.axis_index("c")
    sid = lax.axis_index("s")
    wid = sid * 2 + cid
    r0 = sid * RPT
    pltpu.sync_copy(z64_hbm.at[pl.ds(r0, RPT)], acc_sh.at[pl.ds(r0, RPT)])
    pltpu.sync_copy(src_hbm.at[wid], src_v)
    pltpu.sync_copy(dst_hbm.at[wid], dst_v)
    plsc.subcore_barrier()

    @pl.loop(0, NCHUNK)
    def _(j):
        pltpu.async_copy(g_hbm.at[src_v.at[j]], rows_v, sem).wait()
        pltpu.sync_copy(rows_v, acc_sh.at[dst_v.at[j]], add=True)

    plsc.subcore_barrier()

    @pl.when(cid == 0)
    def _():
        pltpu.sync_copy(acc_sh.at[pl.ds(r0, RPT)], acc0_hbm.at[pl.ds(r0, RPT)])

    @pl.when(cid == 1)
    def _():
        pltpu.sync_copy(acc_sh.at[pl.ds(r0, RPT)], acc1_hbm.at[pl.ds(r0, RPT)])


def _tc_a(x_ref, w1_ref, d0_ref, d1_ref, g_ref, dinv_ref):
    deg = 1.0 + d0_ref[:, 0:1] + d1_ref[:, 0:1]
    dinv = lax.rsqrt(jnp.maximum(deg, 1.0))
    dinvb = jnp.broadcast_to(dinv, (BLK, H))
    h = jnp.dot(x_ref[...], w1_ref[...], preferred_element_type=jnp.float32)
    g_ref[...] = h * dinvb
    dinv_ref[...] = dinvb


def _tc_b(a0_ref, a1_ref, g1_ref, dinv_ref, b1_ref, w2_ref, g2_ref):
    dinvb = dinv_ref[...]
    out1 = jnp.maximum(
        dinvb * (a0_ref[...] + a1_ref[...] + g1_ref[...]) + b1_ref[...], 0.0)
    g2_ref[...] = jnp.dot(
        out1, w2_ref[...], preferred_element_type=jnp.float32) * dinvb


def _tc_c(a0_ref, a1_ref, g2_ref, dinv_ref, b2_ref, batch_ref, wc_ref, bc_ref,
          out_ref, psum, pcnt):
    i = pl.program_id(0)

    @pl.when(i == 0)
    def _():
        psum[...] = jnp.zeros_like(psum)
        pcnt[...] = jnp.zeros_like(pcnt)

    dinvb = dinv_ref[...]
    out2 = jnp.maximum(
        dinvb * (a0_ref[...] + a1_ref[...] + g2_ref[...]) + b2_ref[...], 0.0)
    ids = batch_ref[0]
    iota = lax.broadcasted_iota(jnp.int32, (G, BLK), 0)
    onehot = (iota == ids).astype(jnp.float32)
    psum[...] += jnp.dot(onehot, out2, preferred_element_type=jnp.float32)
    pcnt[...] += jnp.dot(onehot, jnp.ones((BLK, 8), jnp.float32),
                         preferred_element_type=jnp.float32)

    @pl.when(i == NB - 1)
    def _():
        pooled = psum[...] / jnp.maximum(pcnt[:, 0:1], 1.0)
        out_ref[...] = jnp.dot(
            pooled, wc_ref[...], preferred_element_type=jnp.float32) + bc_ref[...]


def _row_spec(width):
    return pl.BlockSpec((BLK, width), lambda i: (i, 0))


def _full_spec(shape):
    return pl.BlockSpec(shape, lambda i: tuple(0 for _ in shape))


_tc_a_call = pl.pallas_call(
    _tc_a,
    grid=(NB,),
    in_specs=[_row_spec(D_IN), _full_spec((D_IN, H)),
              _row_spec(DEG_W), _row_spec(DEG_W)],
    out_specs=[_row_spec(H), _row_spec(H)],
    out_shape=[jax.ShapeDtypeStruct((N, H), jnp.float32),
               jax.ShapeDtypeStruct((N, H), jnp.float32)],
)

_tc_b_call = pl.pallas_call(
    _tc_b,
    grid=(NB,),
    in_specs=[_row_spec(H), _row_spec(H), _row_spec(H), _row_spec(H),
              _full_spec((1, H)), _full_spec((H, H))],
    out_specs=_row_spec(H),
    out_shape=jax.ShapeDtypeStruct((N, H), jnp.float32),
)

_tc_c_call = pl.pallas_call(
    _tc_c,
    grid=(NB,),
    in_specs=[_row_spec(H), _row_spec(H), _row_spec(H), _row_spec(H),
              _full_spec((1, H)),
              pl.BlockSpec((1, 1, BLK), lambda i: (i, 0, 0)),
              _full_spec((H, OUT)), _full_spec((1, OUT))],
    out_specs=_full_spec((G, OUT)),
    out_shape=jax.ShapeDtypeStruct((G, OUT), jnp.float32),
    scratch_shapes=[pltpu.VMEM((G, H), jnp.float32),
                    pltpu.VMEM((G, 8), jnp.float32)],
)


@jax.jit
def kernel(x, edge_index, batch, W1, b1, W2, b2, Wc, bc):
    src = edge_index[0]
    dst = edge_index[1]
    pad = E_PAD - E
    src3 = jnp.concatenate([src, jnp.zeros((pad,), jnp.int32)]).reshape(
        NTILES, NCHUNK, CHUNK)
    dst3 = jnp.concatenate([dst, jnp.full((pad,), N, jnp.int32)]).reshape(
        NTILES, NCHUNK, CHUNK)
    z16 = jnp.zeros((N_PAD, DEG_W), jnp.float32)
    z64 = jnp.zeros((N_PAD, H), jnp.float32)
    batch3 = batch.reshape(NB, 1, BLK)

    deg0, deg1 = _sc_degree_call()(dst3, z16)
    g1, dinvb = _tc_a_call(x, W1, deg0, deg1)
    a0, a1 = _sc_edge_call()(src3, dst3, g1, z64)
    g2 = _tc_b_call(a0[:N], a1[:N], g1, dinvb, b1.reshape(1, H), W2)
    c0, c1 = _sc_edge_call()(src3, dst3, g2, z64)
    return _tc_c_call(c0[:N], c1[:N], g2, dinvb, b2.reshape(1, H), batch3,
                      Wc, bc.reshape(1, OUT))


# speedup vs baseline: 15.9271x; 1.1464x over previous
import functools

import jax
import jax.numpy as jnp
from jax import lax
from jax.experimental import pallas as pl
from jax.experimental.pallas import tpu as pltpu
from jax.experimental.pallas import tpu_sc as plsc

N = 10000
E = 320000
D_IN = 128
H = 64
OUT = 2
G = 128

NTILES = 32
CHUNK = 128
NCHUNK = 80
E_PAD = NTILES * NCHUNK * CHUNK
N_PAD = 10112
RPT = N_PAD // 16
DEG_W = 16

BLK = 1000
NB = N // BLK


@functools.cache
def _sc_degree_call():
    mesh = plsc.VectorSubcoreMesh(core_axis_name="c", subcore_axis_name="s")
    return pl.kernel(
        _sc_degree,
        out_type=[jax.ShapeDtypeStruct((N_PAD, DEG_W), jnp.float32),
                  jax.ShapeDtypeStruct((N_PAD, DEG_W), jnp.float32)],
        mesh=mesh,
        scratch_types=[
            pltpu.VMEM((NCHUNK, CHUNK), jnp.int32),
            pltpu.VMEM((CHUNK, DEG_W), jnp.float32),
            pltpu.VMEM_SHARED((N_PAD, DEG_W), jnp.float32),
        ],
        compiler_params=pltpu.CompilerParams(use_tc_tiling_on_sc=False),
    )


def _sc_degree(dst_hbm, z16_hbm, deg0_hbm, deg1_hbm, dst_v, ones_v, acc_sh):
    cid = lax.axis_index("c")
    sid = lax.axis_index("s")
    wid = sid * 2 + cid
    r0 = sid * RPT
    for r in range(CHUNK):
        ones_v[r] = jnp.ones((16,), jnp.float32)
    pltpu.sync_copy(z16_hbm.at[pl.ds(r0, RPT)], acc_sh.at[pl.ds(r0, RPT)])
    pltpu.sync_copy(dst_hbm.at[wid], dst_v)
    plsc.subcore_barrier()

    @pl.loop(0, NCHUNK)
    def _(j):
        pltpu.sync_copy(ones_v, acc_sh.at[dst_v.at[j]], add=True)

    plsc.subcore_barrier()

    @pl.when(cid == 0)
    def _():
        pltpu.sync_copy(acc_sh.at[pl.ds(r0, RPT)], deg0_hbm.at[pl.ds(r0, RPT)])

    @pl.when(cid == 1)
    def _():
        pltpu.sync_copy(acc_sh.at[pl.ds(r0, RPT)], deg1_hbm.at[pl.ds(r0, RPT)])


NBUF = 4


@functools.cache
def _sc_edge_call():
    mesh = plsc.VectorSubcoreMesh(core_axis_name="c", subcore_axis_name="s")
    return pl.kernel(
        _sc_edge,
        out_type=[jax.ShapeDtypeStruct((N_PAD, H), jnp.float32),
                  jax.ShapeDtypeStruct((N_PAD, H), jnp.float32)],
        mesh=mesh,
        scratch_types=[
            pltpu.VMEM((NCHUNK, CHUNK), jnp.int32),
            pltpu.VMEM((NCHUNK, CHUNK), jnp.int32),
            pltpu.VMEM((NBUF, CHUNK, H), jnp.float32),
            pltpu.VMEM_SHARED((N_PAD, H), jnp.float32),
        ] + [pltpu.SemaphoreType.DMA] * NBUF,
        compiler_params=pltpu.CompilerParams(use_tc_tiling_on_sc=False),
    )


def _sc_edge(src_hbm, dst_hbm, g_hbm, z64_hbm, acc0_hbm, acc1_hbm,
             src_v, dst_v, rows_v, acc_sh, *sems):
    cid = lax.axis_index("c")
    sid = lax.axis_index("s")
    wid = sid * 2 + cid
    r0 = sid * RPT
    pltpu.sync_copy(z64_hbm.at[pl.ds(r0, RPT)], acc_sh.at[pl.ds(r0, RPT)])
    pltpu.sync_copy(src_hbm.at[wid], src_v)
    pltpu.sync_copy(dst_hbm.at[wid], dst_v)
    plsc.subcore_barrier()

    for b in range(NBUF):
        pltpu.async_copy(g_hbm.at[src_v.at[b]], rows_v.at[b], sems[b])

    @pl.loop(0, NCHUNK // NBUF - 1)
    def _(j):
        c0 = j * NBUF
        for b in range(NBUF):
            c = c0 + b
            pltpu.make_async_copy(
                g_hbm.at[src_v.at[c]], rows_v.at[b], sems[b]).wait()
            pltpu.sync_copy(rows_v.at[b], acc_sh.at[dst_v.at[c]], add=True)
            pltpu.async_copy(
                g_hbm.at[src_v.at[c + NBUF]], rows_v.at[b], sems[b])

    for b in range(NBUF):
        c = NCHUNK - NBUF + b
        pltpu.make_async_copy(
            g_hbm.at[src_v.at[c]], rows_v.at[b], sems[b]).wait()
        pltpu.sync_copy(rows_v.at[b], acc_sh.at[dst_v.at[c]], add=True)

    plsc.subcore_barrier()

    @pl.when(cid == 0)
    def _():
        pltpu.sync_copy(acc_sh.at[pl.ds(r0, RPT)], acc0_hbm.at[pl.ds(r0, RPT)])

    @pl.when(cid == 1)
    def _():
        pltpu.sync_copy(acc_sh.at[pl.ds(r0, RPT)], acc1_hbm.at[pl.ds(r0, RPT)])


def _tc_a(x_ref, w1_ref, d0_ref, d1_ref, g_ref, dinv_ref):
    deg = 1.0 + d0_ref[:, 0:1] + d1_ref[:, 0:1]
    dinv = lax.rsqrt(jnp.maximum(deg, 1.0))
    dinvb = jnp.broadcast_to(dinv, (BLK, H))
    h = jnp.dot(x_ref[...], w1_ref[...], preferred_element_type=jnp.float32)
    g_ref[...] = h * dinvb
    dinv_ref[...] = dinvb


def _tc_b(a0_ref, a1_ref, g1_ref, dinv_ref, b1_ref, w2_ref, g2_ref):
    dinvb = dinv_ref[...]
    out1 = jnp.maximum(
        dinvb * (a0_ref[...] + a1_ref[...] + g1_ref[...]) + b1_ref[...], 0.0)
    g2_ref[...] = jnp.dot(
        out1, w2_ref[...], preferred_element_type=jnp.float32) * dinvb


def _tc_c(a0_ref, a1_ref, g2_ref, dinv_ref, b2_ref, batch_ref, wc_ref, bc_ref,
          out_ref, psum, pcnt):
    i = pl.program_id(0)

    @pl.when(i == 0)
    def _():
        psum[...] = jnp.zeros_like(psum)
        pcnt[...] = jnp.zeros_like(pcnt)

    dinvb = dinv_ref[...]
    out2 = jnp.maximum(
        dinvb * (a0_ref[...] + a1_ref[...] + g2_ref[...]) + b2_ref[...], 0.0)
    ids = batch_ref[0]
    iota = lax.broadcasted_iota(jnp.int32, (G, BLK), 0)
    onehot = (iota == ids).astype(jnp.float32)
    psum[...] += jnp.dot(onehot, out2, preferred_element_type=jnp.float32)
    pcnt[...] += jnp.dot(onehot, jnp.ones((BLK, 8), jnp.float32),
                         preferred_element_type=jnp.float32)

    @pl.when(i == NB - 1)
    def _():
        pooled = psum[...] / jnp.maximum(pcnt[:, 0:1], 1.0)
        out_ref[...] = jnp.dot(
            pooled, wc_ref[...], preferred_element_type=jnp.float32) + bc_ref[...]


def _row_spec(width):
    return pl.BlockSpec((BLK, width), lambda i: (i, 0))


def _full_spec(shape):
    return pl.BlockSpec(shape, lambda i: tuple(0 for _ in shape))


_tc_a_call = pl.pallas_call(
    _tc_a,
    grid=(NB,),
    in_specs=[_row_spec(D_IN), _full_spec((D_IN, H)),
              _row_spec(DEG_W), _row_spec(DEG_W)],
    out_specs=[_row_spec(H), _row_spec(H)],
    out_shape=[jax.ShapeDtypeStruct((N, H), jnp.float32),
               jax.ShapeDtypeStruct((N, H), jnp.float32)],
)

_tc_b_call = pl.pallas_call(
    _tc_b,
    grid=(NB,),
    in_specs=[_row_spec(H), _row_spec(H), _row_spec(H), _row_spec(H),
              _full_spec((1, H)), _full_spec((H, H))],
    out_specs=_row_spec(H),
    out_shape=jax.ShapeDtypeStruct((N, H), jnp.float32),
)

_tc_c_call = pl.pallas_call(
    _tc_c,
    grid=(NB,),
    in_specs=[_row_spec(H), _row_spec(H), _row_spec(H), _row_spec(H),
              _full_spec((1, H)),
              pl.BlockSpec((1, 1, BLK), lambda i: (i, 0, 0)),
              _full_spec((H, OUT)), _full_spec((1, OUT))],
    out_specs=_full_spec((G, OUT)),
    out_shape=jax.ShapeDtypeStruct((G, OUT), jnp.float32),
    scratch_shapes=[pltpu.VMEM((G, H), jnp.float32),
                    pltpu.VMEM((G, 8), jnp.float32)],
)


@jax.jit
def kernel(x, edge_index, batch, W1, b1, W2, b2, Wc, bc):
    src = edge_index[0]
    dst = edge_index[1]
    pad = E_PAD - E
    src3 = jnp.concatenate([src, jnp.zeros((pad,), jnp.int32)]).reshape(
        NTILES, NCHUNK, CHUNK)
    dst3 = jnp.concatenate([dst, jnp.full((pad,), N, jnp.int32)]).reshape(
        NTILES, NCHUNK, CHUNK)
    z16 = jnp.zeros((N_PAD, DEG_W), jnp.float32)
    z64 = jnp.zeros((N_PAD, H), jnp.float32)
    batch3 = batch.reshape(NB, 1, BLK)

    deg0, deg1 = _sc_degree_call()(dst3, z16)
    g1, dinvb = _tc_a_call(x, W1, deg0, deg1)
    a0, a1 = _sc_edge_call()(src3, dst3, g1, z64)
    g2 = _tc_b_call(a0[:N], a1[:N], g1, dinvb, b1.reshape(1, H), W2)
    c0, c1 = _sc_edge_call()(src3, dst3, g2, z64)
    return _tc_c_call(c0[:N], c1[:N], g2, dinvb, b2.reshape(1, H), batch3,
                      Wc, bc.reshape(1, OUT))


# speedup vs baseline: 16.2361x; 1.0194x over previous
import functools

import jax
import jax.numpy as jnp
from jax import lax
from jax.experimental import pallas as pl
from jax.experimental.pallas import tpu as pltpu
from jax.experimental.pallas import tpu_sc as plsc

N = 10000
E = 320000
D_IN = 128
H = 64
OUT = 2
G = 128

NTILES = 32
CHUNK = 128
NCHUNK = 80
E_PAD = NTILES * NCHUNK * CHUNK
N_PAD = 10112
RPT = N_PAD // 16
DEG_W = 16

BLK = 1000
NB = N // BLK


@functools.cache
def _sc_degree_call():
    mesh = plsc.VectorSubcoreMesh(core_axis_name="c", subcore_axis_name="s")
    return pl.kernel(
        _sc_degree,
        out_type=[jax.ShapeDtypeStruct((N_PAD, DEG_W), jnp.float32),
                  jax.ShapeDtypeStruct((N_PAD, DEG_W), jnp.float32)],
        mesh=mesh,
        scratch_types=[
            pltpu.VMEM((NCHUNK, CHUNK), jnp.int32),
            pltpu.VMEM((CHUNK, DEG_W), jnp.float32),
            pltpu.VMEM_SHARED((N_PAD, DEG_W), jnp.float32),
        ],
        compiler_params=pltpu.CompilerParams(use_tc_tiling_on_sc=False),
    )


def _sc_degree(dst_hbm, z16_hbm, deg0_hbm, deg1_hbm, dst_v, ones_v, acc_sh):
    cid = lax.axis_index("c")
    sid = lax.axis_index("s")
    wid = sid * 2 + cid
    r0 = sid * RPT
    for r in range(CHUNK):
        ones_v[r] = jnp.ones((16,), jnp.float32)
    pltpu.sync_copy(z16_hbm, acc_sh.at[pl.ds(r0, RPT)])
    pltpu.sync_copy(dst_hbm.at[wid], dst_v)
    plsc.subcore_barrier()

    @pl.loop(0, NCHUNK)
    def _(j):
        pltpu.sync_copy(ones_v, acc_sh.at[dst_v.at[j]], add=True)

    plsc.subcore_barrier()

    @pl.when(cid == 0)
    def _():
        pltpu.sync_copy(acc_sh.at[pl.ds(r0, RPT)], deg0_hbm.at[pl.ds(r0, RPT)])

    @pl.when(cid == 1)
    def _():
        pltpu.sync_copy(acc_sh.at[pl.ds(r0, RPT)], deg1_hbm.at[pl.ds(r0, RPT)])


NBUF = 4
C0 = 128
C1 = 160 - C0
SMAX = 160
TCH = 2720


@functools.cache
def _sc_edge_call():
    mesh = plsc.VectorSubcoreMesh(core_axis_name="c", subcore_axis_name="s")
    return pl.kernel(
        _sc_edge,
        out_type=[jax.ShapeDtypeStruct((N_PAD, H), jnp.float32),
                  jax.ShapeDtypeStruct((N_PAD, H), jnp.float32)],
        mesh=mesh,
        scratch_types=[
            pltpu.VMEM((SMAX, CHUNK), jnp.int32),
            pltpu.VMEM((SMAX, CHUNK), jnp.int32),
            pltpu.VMEM((NBUF, CHUNK, H), jnp.float32),
            pltpu.VMEM_SHARED((N_PAD, H), jnp.float32),
        ] + [pltpu.SemaphoreType.DMA] * NBUF,
        compiler_params=pltpu.CompilerParams(use_tc_tiling_on_sc=False),
    )


def _sc_edge(src_hbm, dst_hbm, g_hbm, z64_hbm, acc0_hbm, acc1_hbm,
             src_v, dst_v, rows_v, acc_sh, *sems):
    cid = lax.axis_index("c")
    sid = lax.axis_index("s")
    r0 = sid * RPT
    nch = jnp.where(cid == 0, C0, C1)
    base = jnp.where(cid == 0, sid * C0, 16 * C0 + sid * C1)
    pltpu.sync_copy(z64_hbm, acc_sh.at[pl.ds(r0, RPT)])

    @pl.when(nch > 0)
    def _():
        pltpu.sync_copy(src_hbm.at[pl.ds(base, SMAX)], src_v)
        pltpu.sync_copy(dst_hbm.at[pl.ds(base, SMAX)], dst_v)

    plsc.subcore_barrier()

    @pl.when(nch > 0)
    def _():
        for b in range(NBUF):
            pltpu.async_copy(g_hbm.at[src_v.at[b]], rows_v.at[b], sems[b])

        @pl.loop(0, nch // NBUF - 1)
        def _(j):
            c0 = j * NBUF
            for b in range(NBUF):
                c = c0 + b
                pltpu.make_async_copy(
                    g_hbm.at[src_v.at[c]], rows_v.at[b], sems[b]).wait()
                pltpu.sync_copy(rows_v.at[b], acc_sh.at[dst_v.at[c]], add=True)
                pltpu.async_copy(
                    g_hbm.at[src_v.at[c + NBUF]], rows_v.at[b], sems[b])

        for b in range(NBUF):
            c = nch - NBUF + b
            pltpu.make_async_copy(
                g_hbm.at[src_v.at[c]], rows_v.at[b], sems[b]).wait()
            pltpu.sync_copy(rows_v.at[b], acc_sh.at[dst_v.at[c]], add=True)

    plsc.subcore_barrier()

    @pl.when(cid == 0)
    def _():
        pltpu.sync_copy(acc_sh.at[pl.ds(r0, RPT)], acc0_hbm.at[pl.ds(r0, RPT)])

    @pl.when(cid == 1)
    def _():
        pltpu.sync_copy(acc_sh.at[pl.ds(r0, RPT)], acc1_hbm.at[pl.ds(r0, RPT)])


def _tc_a(x_ref, w1_ref, d0_ref, d1_ref, g_ref, dinv_ref):
    deg = 1.0 + d0_ref[:, 0:1] + d1_ref[:, 0:1]
    dinv = lax.rsqrt(jnp.maximum(deg, 1.0))
    dinvb = jnp.broadcast_to(dinv, (BLK, H))
    h = jnp.dot(x_ref[...], w1_ref[...], preferred_element_type=jnp.float32)
    g_ref[...] = h * dinvb
    dinv_ref[...] = dinvb


def _tc_b(a0_ref, a1_ref, g1_ref, dinv_ref, b1_ref, w2_ref, g2_ref):
    dinvb = dinv_ref[...]
    out1 = jnp.maximum(
        dinvb * (a0_ref[...] + a1_ref[...] + g1_ref[...]) + b1_ref[...], 0.0)
    g2_ref[...] = jnp.dot(
        out1, w2_ref[...], preferred_element_type=jnp.float32) * dinvb


def _tc_c(a0_ref, a1_ref, g2_ref, dinv_ref, b2_ref, batch_ref, wc_ref, bc_ref,
          out_ref, psum, pcnt):
    i = pl.program_id(0)

    @pl.when(i == 0)
    def _():
        psum[...] = jnp.zeros_like(psum)
        pcnt[...] = jnp.zeros_like(pcnt)

    dinvb = dinv_ref[...]
    out2 = jnp.maximum(
        dinvb * (a0_ref[...] + a1_ref[...] + g2_ref[...]) + b2_ref[...], 0.0)
    ids = batch_ref[0]
    iota = lax.broadcasted_iota(jnp.int32, (G, BLK), 0)
    onehot = (iota == ids).astype(jnp.float32)
    psum[...] += jnp.dot(onehot, out2, preferred_element_type=jnp.float32)
    pcnt[...] += jnp.dot(onehot, jnp.ones((BLK, 8), jnp.float32),
                         preferred_element_type=jnp.float32)

    @pl.when(i == NB - 1)
    def _():
        pooled = psum[...] / jnp.maximum(pcnt[:, 0:1], 1.0)
        out_ref[...] = jnp.dot(
            pooled, wc_ref[...], preferred_element_type=jnp.float32) + bc_ref[...]


def _row_spec(width):
    return pl.BlockSpec((BLK, width), lambda i: (i, 0))


def _full_spec(shape):
    return pl.BlockSpec(shape, lambda i: tuple(0 for _ in shape))


_tc_a_call = pl.pallas_call(
    _tc_a,
    grid=(NB,),
    in_specs=[_row_spec(D_IN), _full_spec((D_IN, H)),
              _row_spec(DEG_W), _row_spec(DEG_W)],
    out_specs=[_row_spec(H), _row_spec(H)],
    out_shape=[jax.ShapeDtypeStruct((N, H), jnp.float32),
               jax.ShapeDtypeStruct((N, H), jnp.float32)],
)

_tc_b_call = pl.pallas_call(
    _tc_b,
    grid=(NB,),
    in_specs=[_row_spec(H), _row_spec(H), _row_spec(H), _row_spec(H),
              _full_spec((1, H)), _full_spec((H, H))],
    out_specs=_row_spec(H),
    out_shape=jax.ShapeDtypeStruct((N, H), jnp.float32),
)

_tc_c_call = pl.pallas_call(
    _tc_c,
    grid=(NB,),
    in_specs=[_row_spec(H), _row_spec(H), _row_spec(H), _row_spec(H),
              _full_spec((1, H)),
              pl.BlockSpec((1, 1, BLK), lambda i: (i, 0, 0)),
              _full_spec((H, OUT)), _full_spec((1, OUT))],
    out_specs=_full_spec((G, OUT)),
    out_shape=jax.ShapeDtypeStruct((G, OUT), jnp.float32),
    scratch_shapes=[pltpu.VMEM((G, H), jnp.float32),
                    pltpu.VMEM((G, 8), jnp.float32)],
)


@jax.jit
def kernel(x, edge_index, batch, W1, b1, W2, b2, Wc, bc):
    src = edge_index[0]
    dst = edge_index[1]
    pad = TCH * CHUNK - E
    src2 = jnp.concatenate([src, jnp.zeros((pad,), jnp.int32)]).reshape(
        TCH, CHUNK)
    dst2 = jnp.concatenate([dst, jnp.full((pad,), N, jnp.int32)]).reshape(
        TCH, CHUNK)
    src3 = src2[:E_PAD // CHUNK].reshape(NTILES, NCHUNK, CHUNK)
    dst3 = dst2[:E_PAD // CHUNK].reshape(NTILES, NCHUNK, CHUNK)
    z16 = jnp.zeros((RPT, DEG_W), jnp.float32)
    z64 = jnp.zeros((RPT, H), jnp.float32)
    batch3 = batch.reshape(NB, 1, BLK)

    deg0, deg1 = _sc_degree_call()(dst3, z16)
    g1, dinvb = _tc_a_call(x, W1, deg0, deg1)
    a0, a1 = _sc_edge_call()(src2, dst2, g1, z64)
    g2 = _tc_b_call(a0[:N], a1[:N], g1, dinvb, b1.reshape(1, H), W2)
    c0, c1 = _sc_edge_call()(src2, dst2, g2, z64)
    return _tc_c_call(c0[:N], c1[:N], g2, dinvb, b2.reshape(1, H), batch3,
                      Wc, bc.reshape(1, OUT))
